# baseline (speedup 1.0000x reference)
import jax
import jax.numpy as jnp
from jax import lax
from jax.experimental import pallas as pl
from jax.experimental.pallas import tpu as pltpu
from jax.experimental.pallas import tpu_sc as plsc

B = 16384
L = 200
DIM = 32
NC = 2
NS = 16
NW = NC * NS
RPW = B // NW
CH = 8
NCHUNK = RPW // CH
CIDX = CH * L
GSZ = [128] * 12 + [64]
LANES = 16


def _sc_pool(idx_hbm, table_hbm, out_hbm, idx_v, rows_v, pooled_v, gs0, gs1):
    wid = lax.axis_index("s") * NC + lax.axis_index("c")
    row0 = wid * RPW
    gsems = (gs0, gs1)

    def stage_idx(c, slot):
        pltpu.sync_copy(idx_hbm.at[pl.ds(row0 + c * CH, CH), :],
                        idx_v.at[slot])

    def fire(slot):
        for r in range(CH):
            for off, sz in ((0, 128), (128, L - 128)):
                pltpu.async_copy(
                    table_hbm.at[idx_v.at[slot, r, pl.ds(off, sz)]],
                    rows_v.at[slot, pl.ds(r * L + off, sz)],
                    gsems[slot])

    def drain(slot):
        pltpu.make_async_copy(
            table_hbm.at[pl.ds(0, CIDX)],
            rows_v.at[slot],
            gsems[slot]).wait()

    def accum(c, slot):
        for r in range(CH):
            def body(j, acc, r=r):
                a0, a1, a2, a3 = acc
                off = r * L + j * 8
                for u in range(0, 8, 2):
                    a0 = a0 + rows_v[slot, off + u, pl.ds(0, LANES)]
                    a1 = a1 + rows_v[slot, off + u, pl.ds(LANES, LANES)]
                    a2 = a2 + rows_v[slot, off + u + 1, pl.ds(0, LANES)]
                    a3 = a3 + rows_v[slot, off + u + 1, pl.ds(LANES, LANES)]
                return (a0, a1, a2, a3)

            z = jnp.zeros((LANES,), jnp.float32)
            a0, a1, a2, a3 = lax.fori_loop(0, L // 8, body, (z, z, z, z))
            prow = c * CH + r
            pooled_v[prow, pl.ds(0, LANES)] = a0 + a2
            pooled_v[prow, pl.ds(LANES, LANES)] = a1 + a3

    stage_idx(0, 0)
    stage_idx(1, 1)
    fire(0)

    def step(i, carry):
        c0 = 2 * i
        c1 = c0 + 1
        not_last = i < NCHUNK // 2 - 1
        fire(1)
        drain(0)

        @pl.when(not_last)
        def _():
            stage_idx(c0 + 2, 0)

        accum(c0, 0)

        @pl.when(not_last)
        def _():
            fire(0)

        drain(1)

        @pl.when(not_last)
        def _():
            stage_idx(c1 + 2, 1)

        accum(c1, 1)
        return carry

    lax.fori_loop(0, NCHUNK // 2, step, 0)
    pltpu.sync_copy(pooled_v, out_hbm.at[pl.ds(row0, RPW), :])


_sc_pool_call = pl.kernel(
    _sc_pool,
    out_type=jax.ShapeDtypeStruct((B, DIM), jnp.float32),
    mesh=plsc.VectorSubcoreMesh(core_axis_name="c", subcore_axis_name="s"),
    scratch_types=[
        pltpu.VMEM((2, CH, L), jnp.int32),
        pltpu.VMEM((2, CIDX, DIM), jnp.float32),
        pltpu.VMEM((RPW, DIM), jnp.float32),
        pltpu.SemaphoreType.DMA,
        pltpu.SemaphoreType.DMA,
    ],
    compiler_params=pltpu.CompilerParams(use_tc_tiling_on_sc=False),
)


def _tc_tail(pooled_ref, wt_ref, b_ref, out_ref):
    x = pooled_ref[...]
    act = jnp.maximum(x, 0.0) + jnp.log1p(jnp.exp(-jnp.abs(x)))
    out_ref[...] = (jnp.sum(act * wt_ref[...], axis=1, keepdims=True)
                    + b_ref[...])


def kernel(inputs, table, W, b):
    pooled = _sc_pool_call(inputs.astype(jnp.int32), table)
    wt = W.reshape(1, DIM)
    out = pl.pallas_call(
        _tc_tail,
        out_shape=jax.ShapeDtypeStruct((B, 1), jnp.float32),
    )(pooled, wt, b)
    return out


# speedup vs baseline: 1.1378x; 1.1378x over previous
import jax
import jax.numpy as jnp
from jax import lax
from jax.experimental import pallas as pl
from jax.experimental.pallas import tpu as pltpu
from jax.experimental.pallas import tpu_sc as plsc

B = 16384
L = 200
DIM = 32
VOCAB = 1000000
NC = 2
NS = 16
NW = NC * NS
RPW = B // NW
CH = 8
NCHUNK = RPW // CH
CIDX = CH * L
GSZ = [128] * 12 + [64]
LANES = 16


def _sc_pool(idx_hbm, table_hbm, out_hbm, idx_v, rows_v, pooled_v, gs0, gs1):
    wid = lax.axis_index("s") * NC + lax.axis_index("c")
    row0 = wid * RPW
    gsems = (gs0, gs1)

    def stage_idx(c, slot):
        pltpu.sync_copy(idx_hbm.at[pl.ds(row0 + c * CH, CH), :],
                        idx_v.at[slot])

    def fire(slot):
        for r in range(CH):
            for off, sz in ((0, 128), (128, L - 128)):
                pltpu.async_copy(
                    table_hbm.at[idx_v.at[slot, r, pl.ds(off, sz)]],
                    rows_v.at[slot, pl.ds(r * L + off, sz)],
                    gsems[slot])

    def drain(slot):
        pltpu.make_async_copy(
            table_hbm.at[pl.ds(0, CIDX)],
            rows_v.at[slot],
            gsems[slot]).wait()

    def accum(c, slot):
        for r in range(CH):
            def body(j, acc, r=r):
                a0, a1, a2, a3 = acc
                off = r * L + j * 8
                for u in range(0, 8, 2):
                    a0 = a0 + rows_v[slot, off + u, pl.ds(0, LANES)]
                    a1 = a1 + rows_v[slot, off + u, pl.ds(LANES, LANES)]
                    a2 = a2 + rows_v[slot, off + u + 1, pl.ds(0, LANES)]
                    a3 = a3 + rows_v[slot, off + u + 1, pl.ds(LANES, LANES)]
                return (a0, a1, a2, a3)

            z = jnp.zeros((LANES,), jnp.float32)
            a0, a1, a2, a3 = lax.fori_loop(0, L // 8, body, (z, z, z, z))
            prow = c * CH + r
            pooled_v[prow, pl.ds(0, LANES)] = a0 + a2
            pooled_v[prow, pl.ds(LANES, LANES)] = a1 + a3

    stage_idx(0, 0)
    stage_idx(1, 1)
    fire(0)

    def step(i, carry):
        c0 = 2 * i
        c1 = c0 + 1
        not_last = i < NCHUNK // 2 - 1
        fire(1)
        drain(0)

        @pl.when(not_last)
        def _():
            stage_idx(c0 + 2, 0)

        accum(c0, 0)

        @pl.when(not_last)
        def _():
            fire(0)

        drain(1)

        @pl.when(not_last)
        def _():
            stage_idx(c1 + 2, 1)

        accum(c1, 1)
        return carry

    lax.fori_loop(0, NCHUNK // 2, step, 0)
    pltpu.sync_copy(pooled_v, out_hbm.at[pl.ds(row0, RPW), :])


_sc_pool_call = pl.kernel(
    _sc_pool,
    out_type=jax.ShapeDtypeStruct((B, DIM), jnp.float32),
    mesh=plsc.VectorSubcoreMesh(core_axis_name="c", subcore_axis_name="s"),
    scratch_types=[
        pltpu.VMEM((2, CH, L), jnp.int32),
        pltpu.VMEM((2, CIDX, DIM), jnp.float32),
        pltpu.VMEM((RPW, DIM), jnp.float32),
        pltpu.SemaphoreType.DMA,
        pltpu.SemaphoreType.DMA,
    ],
    compiler_params=pltpu.CompilerParams(use_tc_tiling_on_sc=False),
)


TBK = 4096
TOR = TBK // 4
NBLK = (VOCAB + TBK - 1) // TBK


def _tc_detile(tt_ref, out_ref):
    xt = tt_ref[...].T
    xt4 = xt.reshape(TOR, 4, DIM)
    out_ref[...] = jnp.concatenate([xt4[:, a, :] for a in range(4)], axis=1)


_detile_call = pl.pallas_call(
    _tc_detile,
    grid=(NBLK,),
    in_specs=[pl.BlockSpec((DIM, TBK), lambda k: (0, k))],
    out_specs=pl.BlockSpec((TOR, 128), lambda k: (k, 0)),
    out_shape=jax.ShapeDtypeStruct((VOCAB * DIM // 128, 128), jnp.float32),
)


def _tc_tail(pooled_ref, wt_ref, b_ref, out_ref):
    x = pooled_ref[...]
    act = jnp.maximum(x, 0.0) + jnp.log1p(jnp.exp(-jnp.abs(x)))
    out_ref[...] = (jnp.sum(act * wt_ref[...], axis=1, keepdims=True)
                    + b_ref[...])


def kernel(inputs, table, W, b):
    table_lin = _detile_call(table.T).reshape(VOCAB, DIM)
    pooled = _sc_pool_call(inputs.astype(jnp.int32), table_lin)
    wt = W.reshape(1, DIM)
    out = pl.pallas_call(
        _tc_tail,
        out_shape=jax.ShapeDtypeStruct((B, 1), jnp.float32),
    )(pooled, wt, b)
    return out


# speedup vs baseline: 1.2399x; 1.0898x over previous
import jax
import jax.numpy as jnp
from jax import lax
from jax.experimental import pallas as pl
from jax.experimental.pallas import tpu as pltpu
from jax.experimental.pallas import tpu_sc as plsc

B = 16384
L = 200
DIM = 32
VOCAB = 1000000
NC = 2
NS = 16
NW = NC * NS
RPW = B // NW
CH = 8
NCHUNK = RPW // CH
CIDX = CH * L
GSZ = [128] * 12 + [64]
LANES = 16


def _sc_pool(idx_hbm, table_hbm, out_hbm, idx_v, rows_v, pooled_v, gs0, gs1):
    wid = lax.axis_index("s") * NC + lax.axis_index("c")
    row0 = wid * RPW
    gsems = (gs0, gs1)

    def stage_idx(c, slot):
        pltpu.sync_copy(idx_hbm.at[pl.ds(row0 + c * CH, CH), :],
                        idx_v.at[slot])

    def fire(slot):
        for r in range(CH):
            for off, sz in ((0, 128), (128, L - 128)):
                pltpu.async_copy(
                    table_hbm.at[idx_v.at[slot, r, pl.ds(off, sz)]],
                    rows_v.at[slot, pl.ds(r * L + off, sz)],
                    gsems[slot])

    def drain(slot):
        pltpu.make_async_copy(
            table_hbm.at[pl.ds(0, CIDX)],
            rows_v.at[slot],
            gsems[slot]).wait()

    def accum(c, slot):
        for r in range(CH):
            def body(j, acc, r=r):
                a0, a1, a2, a3 = acc
                off = r * L + j * 8
                for u in range(0, 8, 2):
                    a0 = a0 + rows_v[slot, off + u, pl.ds(0, LANES)]
                    a1 = a1 + rows_v[slot, off + u, pl.ds(LANES, LANES)]
                    a2 = a2 + rows_v[slot, off + u + 1, pl.ds(0, LANES)]
                    a3 = a3 + rows_v[slot, off + u + 1, pl.ds(LANES, LANES)]
                return (a0, a1, a2, a3)

            z = jnp.zeros((LANES,), jnp.float32)
            a0, a1, a2, a3 = lax.fori_loop(0, L // 8, body, (z, z, z, z))
            prow = c * CH + r
            pooled_v[prow, pl.ds(0, LANES)] = a0 + a2
            pooled_v[prow, pl.ds(LANES, LANES)] = a1 + a3

    stage_idx(0, 0)
    stage_idx(1, 1)
    fire(0)

    def step(i, carry):
        c0 = 2 * i
        c1 = c0 + 1
        not_last = i < NCHUNK // 2 - 1
        fire(1)
        drain(0)

        @pl.when(not_last)
        def _():
            stage_idx(c0 + 2, 0)

        accum(c0, 0)

        @pl.when(not_last)
        def _():
            fire(0)

        drain(1)

        @pl.when(not_last)
        def _():
            stage_idx(c1 + 2, 1)

        accum(c1, 1)
        return carry

    lax.fori_loop(0, NCHUNK // 2, step, 0)
    pltpu.sync_copy(pooled_v, out_hbm.at[pl.ds(row0, RPW), :])


_sc_pool_call = pl.kernel(
    _sc_pool,
    out_type=jax.ShapeDtypeStruct((B, DIM), jnp.float32),
    mesh=plsc.VectorSubcoreMesh(core_axis_name="c", subcore_axis_name="s"),
    scratch_types=[
        pltpu.VMEM((2, CH, L), jnp.int32),
        pltpu.VMEM((2, CIDX, DIM), jnp.float32),
        pltpu.VMEM((RPW, DIM), jnp.float32),
        pltpu.SemaphoreType.DMA,
        pltpu.SemaphoreType.DMA,
    ],
    compiler_params=pltpu.CompilerParams(use_tc_tiling_on_sc=False),
)


TBK = 16384
TOR = TBK // 4
NBLK = (VOCAB + TBK - 1) // TBK


def _tc_detile(tt_ref, out_ref):
    xt = tt_ref[...].T
    xt4 = xt.reshape(TOR, 4, DIM)
    for a in range(4):
        out_ref[:, pl.ds(DIM * a, DIM)] = xt4[:, a, :]


_detile_call = pl.pallas_call(
    _tc_detile,
    grid=(NBLK,),
    in_specs=[pl.BlockSpec((DIM, TBK), lambda k: (0, k))],
    out_specs=pl.BlockSpec((TOR, 128), lambda k: (k, 0)),
    out_shape=jax.ShapeDtypeStruct((VOCAB * DIM // 128, 128), jnp.float32),
)


def _tc_tail(pooled_ref, wt_ref, b_ref, out_ref):
    x = pooled_ref[...]
    act = jnp.maximum(x, 0.0) + jnp.log1p(jnp.exp(-jnp.abs(x)))
    out_ref[...] = (jnp.sum(act * wt_ref[...], axis=1, keepdims=True)
                    + b_ref[...])


def kernel(inputs, table, W, b):
    table_lin = _detile_call(table.T).reshape(VOCAB, DIM)
    pooled = _sc_pool_call(inputs.astype(jnp.int32), table_lin)
    wt = W.reshape(1, DIM)
    out = pl.pallas_call(
        _tc_tail,
        out_shape=jax.ShapeDtypeStruct((B, 1), jnp.float32),
    )(pooled, wt, b)
    return out
